# baseline (speedup 1.0000x reference)
import functools

import jax
import jax.numpy as jnp
from jax import lax
from jax.experimental import pallas as pl
from jax.experimental.pallas import tpu as pltpu
from jax.experimental.pallas import tpu_sc as plsc

NUM_EMB = 1_000_000
DIM = 32
BATCH = 16384
FIELDS = 100
TOTAL = BATCH * FIELDS

NUM_CORES = 2
NUM_SUBCORES = 16
NUM_WORKERS = NUM_CORES * NUM_SUBCORES
CHUNK = BATCH // NUM_WORKERS
NSLOT = 2
LANES = 16


def _build_gather():
    mesh = plsc.VectorSubcoreMesh(core_axis_name="c", subcore_axis_name="s")

    @functools.partial(
        pl.kernel,
        mesh=mesh,
        out_type=jax.ShapeDtypeStruct((FIELDS * DIM * BATCH,), jnp.float32),
        scratch_types=[
            [pltpu.VMEM((CHUNK,), jnp.int32) for _ in range(NSLOT)],
            [pltpu.VMEM((CHUNK, DIM), jnp.float32) for _ in range(NSLOT)],
            [pltpu.VMEM((DIM * CHUNK,), jnp.float32) for _ in range(NSLOT)],
            [pltpu.SemaphoreType.DMA for _ in range(NSLOT)],
            [pltpu.SemaphoreType.DMA for _ in range(NSLOT)],
            [pltpu.SemaphoreType.DMA for _ in range(NSLOT)],
        ],
        compiler_params=pltpu.CompilerParams(
            use_tc_tiling_on_sc=False, needs_layout_passes=False),
    )
    def emb_kernel(idx_hbm, table_hbm, out_hbm, idx_v, rows_v, trans_v,
                   idx_sem, gat_sem, out_sem):
        wid = lax.axis_index("s") * NUM_CORES + lax.axis_index("c")
        bbase = wid * CHUNK

        def issue_idx(f, b):
            pltpu.make_async_copy(
                idx_hbm.at[pl.ds(f * BATCH + bbase, CHUNK)], idx_v[b],
                idx_sem[b]).start()

        def wait_idx(b):
            pltpu.make_async_copy(
                idx_hbm.at[pl.ds(bbase, CHUNK)], idx_v[b], idx_sem[b]).wait()

        def issue_gather(b):
            pltpu.make_async_copy(
                table_hbm.at[idx_v[b]], rows_v[b], gat_sem[b]).start()

        def wait_gather(b):
            pltpu.make_async_copy(
                table_hbm.at[idx_v[b]], rows_v[b], gat_sem[b]).wait()

        def issue_out(f, b):
            for tr in range(DIM // 8):
                pltpu.make_async_copy(
                    trans_v[b].at[pl.ds(tr * 4096, 4096)],
                    out_hbm.at[pl.ds(
                        ((f * (DIM // 8) + tr) * (BATCH // 128)
                         + wid * (CHUNK // 128)) * 1024, 4096)],
                    out_sem[b]).start()

        def wait_out(b):
            pltpu.make_async_copy(
                out_hbm.at[pl.ds(0, DIM * CHUNK)], trans_v[b],
                out_sem[b]).wait()

        lane_iota = lax.iota(jnp.int32, LANES)

        def transpose(b):
            @plsc.parallel_loop(0, CHUNK, unroll=16)
            def j_body(j):
                boff = (j // 128) * 1024 + j % 128
                col_idx = jnp.full((LANES,), boff, jnp.int32)
                for dg in range(DIM // LANES):
                    d = dg * LANES + lane_iota
                    tile_base = (d // 8) * 4096 + (d % 8) * 128
                    vec = rows_v[b][j, pl.ds(dg * LANES, LANES)]
                    plsc.store_scatter(
                        trans_v[b], [tile_base + col_idx], vec)

        issue_idx(0, 0)
        issue_idx(1, 1)
        wait_idx(0)
        issue_gather(0)
        wait_gather(0)
        wait_idx(1)
        issue_gather(1)
        transpose(0)
        issue_out(0, 0)
        issue_idx(2, 0)
        wait_gather(1)
        wait_idx(0)
        issue_gather(0)
        transpose(1)
        issue_out(1, 1)
        issue_idx(3, 1)

        def body(gg, carry):
            for b in range(NSLOT):
                f = gg * NSLOT + b
                b2 = 1 - b
                wait_gather(b)
                wait_idx(b2)
                issue_gather(b2)
                wait_out(b)
                transpose(b)
                issue_out(f, b)
                issue_idx(f + NSLOT, b)
            return carry

        lax.fori_loop(1, FIELDS // NSLOT - 1, body, 0)

        wait_gather(0)
        wait_idx(1)
        issue_gather(1)
        wait_out(0)
        transpose(0)
        issue_out(FIELDS - 2, 0)
        wait_gather(1)
        wait_out(1)
        transpose(1)
        issue_out(FIELDS - 1, 1)
        for b in range(NSLOT):
            wait_out(b)

    return emb_kernel


_emb_kernel = _build_gather()


def kernel(X, W):
    idx = X.astype(jnp.int32).T.reshape(TOTAL)
    out = _emb_kernel(idx, W)
    o5 = out.reshape(FIELDS, DIM // 8, BATCH // 128, 8, 128)
    return o5.transpose(2, 4, 0, 1, 3).reshape(BATCH, FIELDS, DIM)
